# baseline (speedup 1.0000x reference)
import dataclasses
import functools

import jax
import jax.numpy as jnp
from jax import lax
from jax.experimental import pallas as pl
from jax.experimental.pallas import tpu as pltpu
from jax.experimental.pallas import tpu_sc as plsc

N = 10000
E = 320000
D = 128
DE = 16
LANES = 16

NT = 32
NSUB = 16
C = 64
NCH = 158
NCH_PAD = 160
W_IDX = 32
NBLK = 5
E_PAD = NT * NCH * C

AGG_ROWS = 10112
ZROWS = 632
DUMMY = 10104

EP_BLK = 2048
EP_LAST = E // EP_BLK

PERM = []
for _t in range(D // 32):
    PERM.extend(range(32 * _t, 32 * _t + 32, 2))
    PERM.extend(range(32 * _t + 1, 32 * _t + 32, 2))


def _ep_tc(ea_env, ea_act, We0, be0, We1, be1, We2, be2):
    def body(env_ref, act_ref, we0_ref, be0_ref, we1_ref, be1_ref,
             we2_ref, be2_ref, o0_ref, o1_ref, o2_ref):
        def proj(x, w_ref, b_ref):
            y = jnp.dot(x, w_ref[...], preferred_element_type=jnp.float32)
            return (y + b_ref[...]).astype(jnp.bfloat16)

        o0_ref[...] = proj(env_ref[...], we0_ref, be0_ref)
        act = act_ref[...]
        o1_ref[...] = proj(act, we1_ref, be1_ref)
        o2_ref[...] = proj(act, we2_ref, be2_ref)

    clamped = pl.BlockSpec((EP_BLK, DE), lambda i: (jnp.minimum(i, EP_LAST), 0))
    wspec = pl.BlockSpec((DE, D), lambda i: (0, 0))
    bspec = pl.BlockSpec((1, D), lambda i: (0, 0))
    ospec = pl.BlockSpec((EP_BLK, D), lambda i: (i, 0))
    oshape = jax.ShapeDtypeStruct((E_PAD, D), jnp.bfloat16)
    return pl.pallas_call(
        body,
        grid=(E_PAD // EP_BLK,),
        in_specs=[clamped, clamped, wspec, bspec, wspec, bspec, wspec, bspec],
        out_specs=[ospec, ospec, ospec],
        out_shape=[oshape, oshape, oshape],
    )(ea_env, ea_act, We0, be0.reshape(1, D), We1, be1.reshape(1, D),
      We2, be2.reshape(1, D))


def _update_tc(h, aggp, W, Wp, b, do_relu, emit_bf16):
    def body(h_ref, a_ref, w_ref, wp_ref, b_ref, *out_refs):
        ap = a_ref[0, pl.ds(0, N), :] + a_ref[1, pl.ds(0, N), :]
        y = (jnp.dot(h_ref[...], w_ref[...], preferred_element_type=jnp.float32)
             + jnp.dot(ap, wp_ref[...], preferred_element_type=jnp.float32)
             + b_ref[...])
        if do_relu:
            y = jnp.maximum(y, 0.0)
        out_refs[0][...] = y
        if emit_bf16:
            out_refs[1][...] = y.astype(jnp.bfloat16)

    oshape = [jax.ShapeDtypeStruct((N, D), jnp.float32)]
    if emit_bf16:
        oshape.append(jax.ShapeDtypeStruct((N, D), jnp.bfloat16))
    return pl.pallas_call(
        body,
        out_shape=oshape,
    )(h, aggp, W, Wp, b.reshape(1, D))


def _sc_layer(h_bf, ep, src_p, dst_p):
    mesh = plsc.VectorSubcoreMesh(core_axis_name="c", subcore_axis_name="s")
    cp = pltpu.CompilerParams(needs_layout_passes=False,
                              use_tc_tiling_on_sc=False)

    @functools.partial(
        pl.kernel,
        out_type=jax.ShapeDtypeStruct((2, AGG_ROWS, D), jnp.float32),
        mesh=mesh,
        compiler_params=cp,
        scratch_types=[
            pltpu.VMEM((W_IDX, C), jnp.int32),
            pltpu.VMEM((W_IDX, C), jnp.int32),
            pltpu.VMEM((C, D // 2), jnp.int32),
            pltpu.VMEM((C, D // 2), jnp.int32),
            pltpu.VMEM((C, D // 2), jnp.int32),
            pltpu.VMEM((C, D // 2), jnp.int32),
            pltpu.VMEM((C, D), jnp.float32),
            pltpu.VMEM_SHARED((AGG_ROWS, D), jnp.float32),
            pltpu.SemaphoreType.DMA,
            pltpu.SemaphoreType.DMA,
            pltpu.SemaphoreType.DMA,
            pltpu.SemaphoreType.DMA,
        ],
    )
    def k(h_hbm, ep_hbm, src_hbm, dst_hbm, out_hbm,
          src_v, dst_v, hbuf0, hbuf1, epbuf0, epbuf1, mbuf, agg,
          gsem0, gsem1, esem0, esem1):
        cid = lax.axis_index("c")
        sid = lax.axis_index("s")
        wid = cid * NSUB + sid

        def compute(hb, eb):
            @pl.loop(0, C)
            def _(e):
                for t in range(D // (2 * LANES)):
                    sl = pl.ds(t * LANES, LANES)
                    hv = plsc.bitcast(hb[e, sl], jnp.bfloat16)
                    ev = plsc.bitcast(eb[e, sl], jnp.bfloat16)
                    m = jnp.maximum(hv + ev,
                                    jnp.zeros((2 * LANES,), jnp.bfloat16))
                    lo, hi = plsc.unpack(m, format=plsc.PackFormat.INTERLEAVED)
                    mbuf[e, pl.ds(t * 2 * LANES, LANES)] = lo
                    mbuf[e, pl.ds(t * 2 * LANES + LANES, LANES)] = hi

        @pl.loop(0, C)
        def _(i):
            for t in range(D // LANES):
                mbuf[i, pl.ds(t * LANES, LANES)] = jnp.zeros((LANES,), jnp.float32)

        for r in range(9):
            pltpu.sync_copy(mbuf, agg.at[pl.ds(sid * ZROWS + r * C, C)])
        pltpu.sync_copy(mbuf.at[pl.ds(0, ZROWS - 9 * C)],
                        agg.at[pl.ds(sid * ZROWS + 9 * C, ZROWS - 9 * C)])
        plsc.subcore_barrier()

        @pl.loop(0, NBLK)
        def _(blk):
            base = blk * W_IDX
            nin = lax.min(W_IDX, NCH - base)
            npair = nin // 2
            pltpu.sync_copy(src_hbm.at[wid, pl.ds(base, W_IDX)], src_v)
            pltpu.sync_copy(dst_hbm.at[wid, pl.ds(base, W_IDX)], dst_v)
            pltpu.async_copy(h_hbm.at[src_v.at[0]], hbuf0, gsem0)
            pltpu.async_copy(ep_hbm.at[wid, base], epbuf0, esem0)

            @pl.loop(0, npair)
            def _(p):
                a = 2 * p
                b = a + 1
                pltpu.make_async_copy(h_hbm.at[src_v.at[a]], hbuf0, gsem0).wait()
                pltpu.make_async_copy(ep_hbm.at[wid, base + a], epbuf0, esem0).wait()
                pltpu.async_copy(h_hbm.at[src_v.at[b]], hbuf1, gsem1)
                pltpu.async_copy(ep_hbm.at[wid, base + b], epbuf1, esem1)
                compute(hbuf0, epbuf0)
                pltpu.sync_copy(mbuf, agg.at[dst_v.at[a]], add=True)
                pltpu.make_async_copy(h_hbm.at[src_v.at[b]], hbuf1, gsem1).wait()
                pltpu.make_async_copy(ep_hbm.at[wid, base + b], epbuf1, esem1).wait()

                @pl.when(a + 2 < nin)
                def _():
                    pltpu.async_copy(h_hbm.at[src_v.at[a + 2]], hbuf0, gsem0)
                    pltpu.async_copy(ep_hbm.at[wid, base + a + 2], epbuf0, esem0)

                compute(hbuf1, epbuf1)
                pltpu.sync_copy(mbuf, agg.at[dst_v.at[b]], add=True)

        plsc.subcore_barrier()
        pltpu.sync_copy(agg.at[pl.ds(sid * ZROWS, ZROWS)],
                        out_hbm.at[cid, pl.ds(sid * ZROWS, ZROWS)])

    return k(h_bf, ep, src_p, dst_p)


def kernel(x, edge_index, env_edge_attr, act_edge_attr,
           We0, be0, W0, b0, We1, be1, W1, b1, We2, be2, W2, b2):
    pad = E_PAD - E
    src = edge_index[0].astype(jnp.int32)
    dst = edge_index[1].astype(jnp.int32)
    src_p = jnp.concatenate([src, jnp.zeros((pad,), jnp.int32)]).reshape(NT, NCH, C)
    dst_p = jnp.concatenate([dst, jnp.full((pad,), DUMMY, jnp.int32)]).reshape(NT, NCH, C)
    src_p = jnp.pad(src_p, ((0, 0), (0, NCH_PAD - NCH), (0, 0)))
    dst_p = jnp.pad(dst_p, ((0, 0), (0, NCH_PAD - NCH), (0, 0)),
                    constant_values=DUMMY)

    ep0, ep1, ep2 = _ep_tc(env_edge_attr, act_edge_attr,
                           We0, be0, We1, be1, We2, be2)

    def pack_rows(a):
        return lax.bitcast_convert_type(
            a.reshape(*a.shape[:-1], D // 2, 2), jnp.int32)

    ep0 = pack_rows(ep0).reshape(NT, NCH, C, D // 2)
    ep1 = pack_rows(ep1).reshape(NT, NCH, C, D // 2)
    ep2 = pack_rows(ep2).reshape(NT, NCH, C, D // 2)
    perm = jnp.asarray(PERM, dtype=jnp.int32)

    h = x
    h_bf = x.astype(jnp.bfloat16)
    for i, (ep, W, b) in enumerate(((ep0, W0, b0), (ep1, W1, b1),
                                    (ep2, W2, b2))):
        last = i == 2
        aggp = _sc_layer(pack_rows(h_bf), ep, src_p, dst_p)
        outs = _update_tc(h, aggp, W, W[perm, :], b,
                          do_relu=not last, emit_bf16=not last)
        h = outs[0]
        if not last:
            h_bf = outs[1]
    return h


# speedup vs baseline: 1.1091x; 1.1091x over previous
import dataclasses
import functools

import jax
import jax.numpy as jnp
from jax import lax
from jax.experimental import pallas as pl
from jax.experimental.pallas import tpu as pltpu
from jax.experimental.pallas import tpu_sc as plsc

N = 10000
E = 320000
D = 128
DE = 16
LANES = 16

NT = 32
NSUB = 16
C = 64
NCH = 158
NCH_PAD = 160
W_IDX = 32
NBLK = 5
E_PAD = NT * NCH * C

AGG_ROWS = 10112
ZROWS = 632
DUMMY = 10104

EP_BLK = 2048
EP_LAST = E // EP_BLK

ILV = [0] * D
for _t in range(D // 32):
    for _k in range(16):
        ILV[32 * _t + 2 * _k] = 32 * _t + _k
        ILV[32 * _t + 2 * _k + 1] = 32 * _t + 16 + _k


def _ep_tc(ea_env, ea_act, We0, be0, We1, be1, We2, be2):
    def body(env_ref, act_ref, we0_ref, be0_ref, we1_ref, be1_ref,
             we2_ref, be2_ref, o0_ref, o1_ref, o2_ref):
        def proj(x, w_ref, b_ref):
            y = jnp.dot(x, w_ref[...], preferred_element_type=jnp.float32)
            return (y + b_ref[...]).astype(jnp.bfloat16)

        o0_ref[...] = proj(env_ref[...], we0_ref, be0_ref)
        act = act_ref[...]
        o1_ref[...] = proj(act, we1_ref, be1_ref)
        o2_ref[...] = proj(act, we2_ref, be2_ref)

    clamped = pl.BlockSpec((EP_BLK, DE), lambda i: (jnp.minimum(i, EP_LAST), 0))
    wspec = pl.BlockSpec((DE, D), lambda i: (0, 0))
    bspec = pl.BlockSpec((1, D), lambda i: (0, 0))
    ospec = pl.BlockSpec((EP_BLK, D), lambda i: (i, 0))
    oshape = jax.ShapeDtypeStruct((E_PAD, D), jnp.bfloat16)
    return pl.pallas_call(
        body,
        grid=(E_PAD // EP_BLK,),
        in_specs=[clamped, clamped, wspec, bspec, wspec, bspec, wspec, bspec],
        out_specs=[ospec, ospec, ospec],
        out_shape=[oshape, oshape, oshape],
    )(ea_env, ea_act, We0, be0.reshape(1, D), We1, be1.reshape(1, D),
      We2, be2.reshape(1, D))


def _update_tc(h, aggp, W, b, do_relu):
    def body(h_ref, a_ref, w_ref, b_ref, out_ref):
        s = (h_ref[...]
             + a_ref[0, pl.ds(0, N), :]
             + a_ref[1, pl.ds(0, N), :])
        y = jnp.dot(s, w_ref[...], preferred_element_type=jnp.float32) + b_ref[...]
        if do_relu:
            y = jnp.maximum(y, 0.0)
        out_ref[...] = y

    return pl.pallas_call(
        body,
        out_shape=jax.ShapeDtypeStruct((N, D), jnp.float32),
    )(h, aggp, W, b.reshape(1, D))


def _sc_layer(h, ep, src_p, dst_p):
    mesh = plsc.VectorSubcoreMesh(core_axis_name="c", subcore_axis_name="s")
    cp = pltpu.CompilerParams(needs_layout_passes=False)

    @functools.partial(
        pl.kernel,
        out_type=jax.ShapeDtypeStruct((2, AGG_ROWS, D), jnp.float32),
        mesh=mesh,
        compiler_params=cp,
        scratch_types=[
            pltpu.VMEM((W_IDX, C), jnp.int32),
            pltpu.VMEM((W_IDX, C), jnp.int32),
            pltpu.VMEM((C, D), jnp.float32),
            pltpu.VMEM((C, D), jnp.float32),
            pltpu.VMEM((C, D // 2), jnp.int32),
            pltpu.VMEM((C, D // 2), jnp.int32),
            pltpu.VMEM_SHARED((AGG_ROWS, D), jnp.float32),
            pltpu.SemaphoreType.DMA,
            pltpu.SemaphoreType.DMA,
            pltpu.SemaphoreType.DMA,
            pltpu.SemaphoreType.DMA,
        ],
    )
    def k(h_hbm, ep_hbm, src_hbm, dst_hbm, out_hbm,
          src_v, dst_v, hbuf0, hbuf1, epbuf0, epbuf1, agg,
          gsem0, gsem1, esem0, esem1):
        cid = lax.axis_index("c")
        sid = lax.axis_index("s")
        wid = cid * NSUB + sid

        def compute(hb, eb):
            @pl.loop(0, C)
            def _(e):
                for t in range(D // (2 * LANES)):
                    ev = plsc.bitcast(eb[e, pl.ds(t * LANES, LANES)],
                                      jnp.bfloat16)
                    lo, hi = plsc.unpack(ev, format=plsc.PackFormat.INTERLEAVED)
                    sl0 = pl.ds(t * 2 * LANES, LANES)
                    sl1 = pl.ds(t * 2 * LANES + LANES, LANES)
                    hb[e, sl0] = jnp.maximum(hb[e, sl0] + lo, 0.0)
                    hb[e, sl1] = jnp.maximum(hb[e, sl1] + hi, 0.0)

        @pl.loop(0, C)
        def _(i):
            for t in range(D // LANES):
                hbuf0[i, pl.ds(t * LANES, LANES)] = jnp.zeros((LANES,), jnp.float32)

        for r in range(9):
            pltpu.sync_copy(hbuf0, agg.at[pl.ds(sid * ZROWS + r * C, C)])
        pltpu.sync_copy(hbuf0.at[pl.ds(0, ZROWS - 9 * C)],
                        agg.at[pl.ds(sid * ZROWS + 9 * C, ZROWS - 9 * C)])
        plsc.subcore_barrier()

        @pl.loop(0, NBLK)
        def _(blk):
            base = blk * W_IDX
            nin = lax.min(W_IDX, NCH - base)
            npair = nin // 2
            pltpu.sync_copy(src_hbm.at[wid, pl.ds(base, W_IDX)], src_v)
            pltpu.sync_copy(dst_hbm.at[wid, pl.ds(base, W_IDX)], dst_v)
            pltpu.async_copy(h_hbm.at[src_v.at[0]], hbuf0, gsem0)
            pltpu.async_copy(ep_hbm.at[wid, base], epbuf0, esem0)

            @pl.loop(0, npair)
            def _(p):
                a = 2 * p
                b = a + 1
                pltpu.make_async_copy(h_hbm.at[src_v.at[a]], hbuf0, gsem0).wait()
                pltpu.make_async_copy(ep_hbm.at[wid, base + a], epbuf0, esem0).wait()
                pltpu.async_copy(h_hbm.at[src_v.at[b]], hbuf1, gsem1)
                pltpu.async_copy(ep_hbm.at[wid, base + b], epbuf1, esem1)
                compute(hbuf0, epbuf0)
                pltpu.sync_copy(hbuf0, agg.at[dst_v.at[a]], add=True)
                pltpu.make_async_copy(h_hbm.at[src_v.at[b]], hbuf1, gsem1).wait()
                pltpu.make_async_copy(ep_hbm.at[wid, base + b], epbuf1, esem1).wait()

                @pl.when(a + 2 < nin)
                def _():
                    pltpu.async_copy(h_hbm.at[src_v.at[a + 2]], hbuf0, gsem0)
                    pltpu.async_copy(ep_hbm.at[wid, base + a + 2], epbuf0, esem0)

                compute(hbuf1, epbuf1)
                pltpu.sync_copy(hbuf1, agg.at[dst_v.at[b]], add=True)

        plsc.subcore_barrier()
        pltpu.sync_copy(agg.at[pl.ds(sid * ZROWS, ZROWS)],
                        out_hbm.at[cid, pl.ds(sid * ZROWS, ZROWS)])

    return k(h, ep, src_p, dst_p)


def kernel(x, edge_index, env_edge_attr, act_edge_attr,
           We0, be0, W0, b0, We1, be1, W1, b1, We2, be2, W2, b2):
    pad = E_PAD - E
    src = edge_index[0].astype(jnp.int32)
    dst = edge_index[1].astype(jnp.int32)
    src_p = jnp.concatenate([src, jnp.zeros((pad,), jnp.int32)]).reshape(NT, NCH, C)
    dst_p = jnp.concatenate([dst, jnp.full((pad,), DUMMY, jnp.int32)]).reshape(NT, NCH, C)
    src_p = jnp.pad(src_p, ((0, 0), (0, NCH_PAD - NCH), (0, 0)))
    dst_p = jnp.pad(dst_p, ((0, 0), (0, NCH_PAD - NCH), (0, 0)),
                    constant_values=DUMMY)

    ilv = jnp.asarray(ILV, dtype=jnp.int32)
    ep0, ep1, ep2 = _ep_tc(env_edge_attr, act_edge_attr,
                           We0[:, ilv], be0[ilv], We1[:, ilv], be1[ilv],
                           We2[:, ilv], be2[ilv])

    def pack_rows(a):
        return lax.bitcast_convert_type(
            a.reshape(*a.shape[:-1], D // 2, 2), jnp.int32)

    ep0 = pack_rows(ep0).reshape(NT, NCH, C, D // 2)
    ep1 = pack_rows(ep1).reshape(NT, NCH, C, D // 2)
    ep2 = pack_rows(ep2).reshape(NT, NCH, C, D // 2)

    h = x
    for i, (ep, W, b) in enumerate(((ep0, W0, b0), (ep1, W1, b1),
                                    (ep2, W2, b2))):
        last = i == 2
        aggp = _sc_layer(h, ep, src_p, dst_p)
        h = _update_tc(h, aggp, W, b, do_relu=not last)
    return h


# speedup vs baseline: 2.4370x; 2.1973x over previous
import dataclasses
import functools

import jax
import jax.numpy as jnp
from jax import lax
from jax.experimental import pallas as pl
from jax.experimental.pallas import tpu as pltpu
from jax.experimental.pallas import tpu_sc as plsc

N = 10000
E = 320000
D = 128
DE = 16
LANES = 16

NT = 32
NSUB = 16
C = 64
NCH = 158
NCH_PAD = 160
W_IDX = 32
NBLK = 5
E_PAD = NT * NCH * C

AGG_ROWS = 10112
ZROWS = 632
DUMMY = 10104

EP_BLK = 2048
EP_LAST = E // EP_BLK

ILV = [0] * D
for _t in range(D // 32):
    for _k in range(16):
        ILV[32 * _t + 2 * _k] = 32 * _t + _k
        ILV[32 * _t + 2 * _k + 1] = 32 * _t + 16 + _k


def _ep_tc(ea_env, ea_act, We0, be0, We1, be1, We2, be2):
    def body(env_ref, act_ref, we0_ref, be0_ref, we1_ref, be1_ref,
             we2_ref, be2_ref, o0_ref, o1_ref, o2_ref):
        def proj(x, w_ref, b_ref):
            y = jnp.dot(x, w_ref[...], preferred_element_type=jnp.float32)
            return y + b_ref[...]

        o0_ref[...] = proj(env_ref[...], we0_ref, be0_ref)
        act = act_ref[...]
        o1_ref[...] = proj(act, we1_ref, be1_ref)
        o2_ref[...] = proj(act, we2_ref, be2_ref)

    clamped = pl.BlockSpec((EP_BLK, DE), lambda i: (jnp.minimum(i, EP_LAST), 0))
    wspec = pl.BlockSpec((DE, D), lambda i: (0, 0))
    bspec = pl.BlockSpec((1, D), lambda i: (0, 0))
    ospec = pl.BlockSpec((EP_BLK, D), lambda i: (i, 0))
    oshape = jax.ShapeDtypeStruct((E_PAD, D), jnp.float32)
    return pl.pallas_call(
        body,
        grid=(E_PAD // EP_BLK,),
        in_specs=[clamped, clamped, wspec, bspec, wspec, bspec, wspec, bspec],
        out_specs=[ospec, ospec, ospec],
        out_shape=[oshape, oshape, oshape],
    )(ea_env, ea_act, We0, be0.reshape(1, D), We1, be1.reshape(1, D),
      We2, be2.reshape(1, D))


def _update_tc(h, aggp, W, b, do_relu):
    def body(h_ref, a_ref, w_ref, b_ref, out_ref):
        s = (h_ref[...]
             + a_ref[0, pl.ds(0, N), :]
             + a_ref[1, pl.ds(0, N), :])
        y = jnp.dot(s, w_ref[...], preferred_element_type=jnp.float32) + b_ref[...]
        if do_relu:
            y = jnp.maximum(y, 0.0)
        out_ref[...] = y

    return pl.pallas_call(
        body,
        out_shape=jax.ShapeDtypeStruct((N, D), jnp.float32),
    )(h, aggp, W, b.reshape(1, D))


def _sc_layer(h, ep, src_p, dst_p):
    mesh = plsc.VectorSubcoreMesh(core_axis_name="c", subcore_axis_name="s")
    @functools.partial(
        pl.kernel,
        out_type=jax.ShapeDtypeStruct((2, AGG_ROWS, D), jnp.float32),
        mesh=mesh,
        scratch_types=[
            pltpu.VMEM((W_IDX, C), jnp.int32),
            pltpu.VMEM((W_IDX, C), jnp.int32),
            pltpu.VMEM((C, D), jnp.float32),
            pltpu.VMEM((C, D), jnp.float32),
            pltpu.VMEM((C, D), jnp.float32),
            pltpu.VMEM((C, D), jnp.float32),
            pltpu.VMEM_SHARED((AGG_ROWS, D), jnp.float32),
            pltpu.SemaphoreType.DMA,
            pltpu.SemaphoreType.DMA,
            pltpu.SemaphoreType.DMA,
            pltpu.SemaphoreType.DMA,
        ],
    )
    def k(h_hbm, ep_hbm, src_hbm, dst_hbm, out_hbm,
          src_v, dst_v, hbuf0, hbuf1, epbuf0, epbuf1, agg,
          gsem0, gsem1, esem0, esem1):
        cid = lax.axis_index("c")
        sid = lax.axis_index("s")
        wid = cid * NSUB + sid

        def compute(hb, eb):
            @pl.loop(0, C)
            def _(e):
                for t in range(D // LANES):
                    sl = pl.ds(t * LANES, LANES)
                    hb[e, sl] = jnp.maximum(hb[e, sl] + eb[e, sl], 0.0)

        @pl.loop(0, C)
        def _(i):
            for t in range(D // LANES):
                hbuf0[i, pl.ds(t * LANES, LANES)] = jnp.zeros((LANES,), jnp.float32)

        for r in range(9):
            pltpu.sync_copy(hbuf0, agg.at[pl.ds(sid * ZROWS + r * C, C)])
        pltpu.sync_copy(hbuf0.at[pl.ds(0, ZROWS - 9 * C)],
                        agg.at[pl.ds(sid * ZROWS + 9 * C, ZROWS - 9 * C)])
        plsc.subcore_barrier()

        @pl.loop(0, NBLK)
        def _(blk):
            base = blk * W_IDX
            nin = lax.min(W_IDX, NCH - base)
            npair = nin // 2
            pltpu.sync_copy(src_hbm.at[wid, pl.ds(base, W_IDX)], src_v)
            pltpu.sync_copy(dst_hbm.at[wid, pl.ds(base, W_IDX)], dst_v)
            pltpu.async_copy(h_hbm.at[src_v.at[0]], hbuf0, gsem0)
            pltpu.async_copy(ep_hbm.at[wid, base], epbuf0, esem0)

            @pl.loop(0, npair)
            def _(p):
                a = 2 * p
                b = a + 1
                pltpu.make_async_copy(h_hbm.at[src_v.at[a]], hbuf0, gsem0).wait()
                pltpu.make_async_copy(ep_hbm.at[wid, base + a], epbuf0, esem0).wait()
                pltpu.async_copy(h_hbm.at[src_v.at[b]], hbuf1, gsem1)
                pltpu.async_copy(ep_hbm.at[wid, base + b], epbuf1, esem1)
                compute(hbuf0, epbuf0)
                pltpu.sync_copy(hbuf0, agg.at[dst_v.at[a]], add=True)
                pltpu.make_async_copy(h_hbm.at[src_v.at[b]], hbuf1, gsem1).wait()
                pltpu.make_async_copy(ep_hbm.at[wid, base + b], epbuf1, esem1).wait()

                @pl.when(a + 2 < nin)
                def _():
                    pltpu.async_copy(h_hbm.at[src_v.at[a + 2]], hbuf0, gsem0)
                    pltpu.async_copy(ep_hbm.at[wid, base + a + 2], epbuf0, esem0)

                compute(hbuf1, epbuf1)
                pltpu.sync_copy(hbuf1, agg.at[dst_v.at[b]], add=True)

        plsc.subcore_barrier()
        pltpu.sync_copy(agg.at[pl.ds(sid * ZROWS, ZROWS)],
                        out_hbm.at[cid, pl.ds(sid * ZROWS, ZROWS)])

    return k(h, ep, src_p, dst_p)


def kernel(x, edge_index, env_edge_attr, act_edge_attr,
           We0, be0, W0, b0, We1, be1, W1, b1, We2, be2, W2, b2):
    pad = E_PAD - E
    src = edge_index[0].astype(jnp.int32)
    dst = edge_index[1].astype(jnp.int32)
    src_p = jnp.concatenate([src, jnp.zeros((pad,), jnp.int32)]).reshape(NT, NCH, C)
    dst_p = jnp.concatenate([dst, jnp.full((pad,), DUMMY, jnp.int32)]).reshape(NT, NCH, C)
    src_p = jnp.pad(src_p, ((0, 0), (0, NCH_PAD - NCH), (0, 0)))
    dst_p = jnp.pad(dst_p, ((0, 0), (0, NCH_PAD - NCH), (0, 0)),
                    constant_values=DUMMY)

    ep0, ep1, ep2 = _ep_tc(env_edge_attr, act_edge_attr,
                           We0, be0, We1, be1, We2, be2)
    ep0 = ep0.reshape(NT, NCH, C, D)
    ep1 = ep1.reshape(NT, NCH, C, D)
    ep2 = ep2.reshape(NT, NCH, C, D)

    h = x
    for i, (ep, W, b) in enumerate(((ep0, W0, b0), (ep1, W1, b1),
                                    (ep2, W2, b2))):
        last = i == 2
        aggp = _sc_layer(h, ep, src_p, dst_p)
        h = _update_tc(h, aggp, W, b, do_relu=not last)
    return h


# speedup vs baseline: 3.4265x; 1.4060x over previous
import dataclasses
import functools

import jax
import jax.numpy as jnp
from jax import lax
from jax.experimental import pallas as pl
from jax.experimental.pallas import tpu as pltpu
from jax.experimental.pallas import tpu_sc as plsc

N = 10000
E = 320000
D = 128
DE = 16
LANES = 16

NT = 32
NSUB = 16
C = 64
NCHUNKS = E // C
NCH = 158
W_IDX = 32
W_READ = 40
E_PAD = NT * NCH * C

AGG_ROWS = 10112
ZROWS = 632

EP_BLK = 2048
EP_LAST = E // EP_BLK


def _ep_tc(ea_env, ea_act, We0, be0, We1, be1, We2, be2):
    def body(env_ref, act_ref, we0_ref, be0_ref, we1_ref, be1_ref,
             we2_ref, be2_ref, o0_ref, o1_ref, o2_ref):
        def proj(x, w_ref, b_ref):
            y = jnp.dot(x, w_ref[...], preferred_element_type=jnp.float32)
            return y + b_ref[...]

        o0_ref[...] = proj(env_ref[...], we0_ref, be0_ref)
        act = act_ref[...]
        o1_ref[...] = proj(act, we1_ref, be1_ref)
        o2_ref[...] = proj(act, we2_ref, be2_ref)

    clamped = pl.BlockSpec((EP_BLK, DE), lambda i: (jnp.minimum(i, EP_LAST), 0))
    wspec = pl.BlockSpec((DE, D), lambda i: (0, 0))
    bspec = pl.BlockSpec((1, D), lambda i: (0, 0))
    ospec = pl.BlockSpec((EP_BLK, D), lambda i: (i, 0))
    oshape = jax.ShapeDtypeStruct((E_PAD, D), jnp.float32)
    return pl.pallas_call(
        body,
        grid=(E_PAD // EP_BLK,),
        in_specs=[clamped, clamped, wspec, bspec, wspec, bspec, wspec, bspec],
        out_specs=[ospec, ospec, ospec],
        out_shape=[oshape, oshape, oshape],
    )(ea_env, ea_act, We0, be0.reshape(1, D), We1, be1.reshape(1, D),
      We2, be2.reshape(1, D))


def _update_tc(h, aggp, W, b, do_relu):
    def body(h_ref, a_ref, w_ref, b_ref, out_ref):
        s = (h_ref[...]
             + a_ref[0, pl.ds(0, N), :]
             + a_ref[1, pl.ds(0, N), :])
        y = jnp.dot(s, w_ref[...], preferred_element_type=jnp.float32) + b_ref[...]
        if do_relu:
            y = jnp.maximum(y, 0.0)
        out_ref[...] = y

    return pl.pallas_call(
        body,
        out_shape=jax.ShapeDtypeStruct((N, D), jnp.float32),
    )(h, aggp, W, b.reshape(1, D))


def _sc_layer(h, ep, src_p, dst_p):
    mesh = plsc.VectorSubcoreMesh(core_axis_name="c", subcore_axis_name="s")
    @functools.partial(
        pl.kernel,
        out_type=jax.ShapeDtypeStruct((2, AGG_ROWS, D), jnp.float32),
        mesh=mesh,
        scratch_types=[
            pltpu.VMEM((W_READ, C), jnp.int32),
            pltpu.VMEM((W_READ, C), jnp.int32),
            pltpu.VMEM((C, D), jnp.float32),
            pltpu.VMEM((C, D), jnp.float32),
            pltpu.VMEM((C, D), jnp.float32),
            pltpu.VMEM((C, D), jnp.float32),
            pltpu.VMEM_SHARED((AGG_ROWS, D), jnp.float32),
            pltpu.SemaphoreType.DMA,
            pltpu.SemaphoreType.DMA,
            pltpu.SemaphoreType.DMA,
            pltpu.SemaphoreType.DMA,
        ],
    )
    def k(h_hbm, ep_hbm, src_hbm, dst_hbm, out_hbm,
          src_v, dst_v, hbuf0, hbuf1, epbuf0, epbuf1, agg,
          gsem0, gsem1, esem0, esem1):
        cid = lax.axis_index("c")
        sid = lax.axis_index("s")
        wid = cid * NSUB + sid
        row0 = wid * NCH
        nch = lax.min(NCH, NCHUNKS - row0)
        nblk = (nch + W_IDX - 1) // W_IDX

        def compute(hb, eb):
            @pl.loop(0, C)
            def _(e):
                for t in range(D // LANES):
                    sl = pl.ds(t * LANES, LANES)
                    hb[e, sl] = jnp.maximum(hb[e, sl] + eb[e, sl], 0.0)

        @pl.loop(0, C)
        def _(i):
            for t in range(D // LANES):
                hbuf0[i, pl.ds(t * LANES, LANES)] = jnp.zeros((LANES,), jnp.float32)

        for r in range(9):
            pltpu.sync_copy(hbuf0, agg.at[pl.ds(sid * ZROWS + r * C, C)])
        pltpu.sync_copy(hbuf0.at[pl.ds(0, ZROWS - 9 * C)],
                        agg.at[pl.ds(sid * ZROWS + 9 * C, ZROWS - 9 * C)])
        plsc.subcore_barrier()

        @pl.loop(0, nblk)
        def _(blk):
            base = blk * W_IDX
            nin = lax.min(W_IDX, nch - base)
            npair = nin // 2
            gstart = row0 + base
            cstart = (lax.min(gstart, NCHUNKS - W_READ) // 8) * 8
            shift = gstart - cstart
            pltpu.sync_copy(src_hbm.at[pl.ds(cstart, W_READ)], src_v)
            pltpu.sync_copy(dst_hbm.at[pl.ds(cstart, W_READ)], dst_v)
            pltpu.async_copy(h_hbm.at[src_v.at[shift]], hbuf0, gsem0)
            pltpu.async_copy(ep_hbm.at[gstart], epbuf0, esem0)

            @pl.loop(0, npair)
            def _(p):
                a = shift + 2 * p
                b = a + 1
                ga = gstart + 2 * p
                pltpu.make_async_copy(h_hbm.at[src_v.at[a]], hbuf0, gsem0).wait()
                pltpu.make_async_copy(ep_hbm.at[ga], epbuf0, esem0).wait()
                pltpu.async_copy(h_hbm.at[src_v.at[b]], hbuf1, gsem1)
                pltpu.async_copy(ep_hbm.at[ga + 1], epbuf1, esem1)
                compute(hbuf0, epbuf0)
                pltpu.sync_copy(hbuf0, agg.at[dst_v.at[a]], add=True)
                pltpu.make_async_copy(h_hbm.at[src_v.at[b]], hbuf1, gsem1).wait()
                pltpu.make_async_copy(ep_hbm.at[ga + 1], epbuf1, esem1).wait()

                @pl.when(2 * p + 2 < nin)
                def _():
                    pltpu.async_copy(h_hbm.at[src_v.at[a + 2]], hbuf0, gsem0)
                    pltpu.async_copy(ep_hbm.at[ga + 2], epbuf0, esem0)

                compute(hbuf1, epbuf1)
                pltpu.sync_copy(hbuf1, agg.at[dst_v.at[b]], add=True)

        plsc.subcore_barrier()
        pltpu.sync_copy(agg.at[pl.ds(sid * ZROWS, ZROWS)],
                        out_hbm.at[cid, pl.ds(sid * ZROWS, ZROWS)])

    return k(h, ep, src_p, dst_p)


def kernel(x, edge_index, env_edge_attr, act_edge_attr,
           We0, be0, W0, b0, We1, be1, W1, b1, We2, be2, W2, b2):
    src_p = edge_index[0].astype(jnp.int32).reshape(NCHUNKS, C)
    dst_p = edge_index[1].astype(jnp.int32).reshape(NCHUNKS, C)

    ep0, ep1, ep2 = _ep_tc(env_edge_attr, act_edge_attr,
                           We0, be0, We1, be1, We2, be2)
    ep0 = ep0.reshape(E_PAD // C, C, D)
    ep1 = ep1.reshape(E_PAD // C, C, D)
    ep2 = ep2.reshape(E_PAD // C, C, D)

    h = x
    for i, (ep, W, b) in enumerate(((ep0, W0, b0), (ep1, W1, b1),
                                    (ep2, W2, b2))):
        last = i == 2
        aggp = _sc_layer(h, ep, src_p, dst_p)
        h = _update_tc(h, aggp, W, b, do_relu=not last)
    return h


# speedup vs baseline: 3.6277x; 1.0587x over previous
import dataclasses
import functools

import jax
import jax.numpy as jnp
from jax import lax
from jax.experimental import pallas as pl
from jax.experimental.pallas import tpu as pltpu
from jax.experimental.pallas import tpu_sc as plsc

N = 10000
E = 320000
D = 128
DE = 16
LANES = 16

NT = 32
NSUB = 16
C = 64
NCHUNKS = E // C
NCH = 158
W_IDX = 32
W_READ = 40
E_PAD = NT * NCH * C

AGG_ROWS = 10112
ZROWS = 632

EP_BLK = 2048
EP_LAST = E // EP_BLK


def _ep_tc(ea, wbs):
    nproj = len(wbs)

    def body(*refs):
        ea_ref = refs[0]
        x = ea_ref[...]
        for j in range(nproj):
            w_ref, b_ref = refs[1 + 2 * j], refs[2 + 2 * j]
            o_ref = refs[1 + 2 * nproj + j]
            y = jnp.dot(x, w_ref[...], preferred_element_type=jnp.float32)
            o_ref[...] = y + b_ref[...]

    clamped = pl.BlockSpec((EP_BLK, DE), lambda i: (jnp.minimum(i, EP_LAST), 0))
    wspec = pl.BlockSpec((DE, D), lambda i: (0, 0))
    bspec = pl.BlockSpec((1, D), lambda i: (0, 0))
    ospec = pl.BlockSpec((EP_BLK, D), lambda i: (i, 0))
    oshape = jax.ShapeDtypeStruct((E_PAD, D), jnp.float32)
    args = [ea]
    for We, be in wbs:
        args += [We, be.reshape(1, D)]
    outs = pl.pallas_call(
        body,
        grid=(E_PAD // EP_BLK,),
        in_specs=[clamped] + [wspec, bspec] * nproj,
        out_specs=[ospec] * nproj,
        out_shape=[oshape] * nproj,
    )(*args)
    return [o.reshape(E_PAD // C, C, D) for o in outs]


def _update_tc(h, aggp, W, b, do_relu):
    def body(h_ref, a_ref, w_ref, b_ref, out_ref):
        s = (h_ref[...]
             + a_ref[0, pl.ds(0, N), :]
             + a_ref[1, pl.ds(0, N), :])
        y = jnp.dot(s, w_ref[...], preferred_element_type=jnp.float32) + b_ref[...]
        if do_relu:
            y = jnp.maximum(y, 0.0)
        out_ref[...] = y

    return pl.pallas_call(
        body,
        out_shape=jax.ShapeDtypeStruct((N, D), jnp.float32),
    )(h, aggp, W, b.reshape(1, D))


def _sc_layer(h, ep, src_p, dst_p):
    mesh = plsc.VectorSubcoreMesh(core_axis_name="c", subcore_axis_name="s")
    @functools.partial(
        pl.kernel,
        out_type=jax.ShapeDtypeStruct((2, AGG_ROWS, D), jnp.float32),
        mesh=mesh,
        scratch_types=[
            pltpu.VMEM((W_READ, C), jnp.int32),
            pltpu.VMEM((W_READ, C), jnp.int32),
            pltpu.VMEM((C, D), jnp.float32),
            pltpu.VMEM((C, D), jnp.float32),
            pltpu.VMEM((C, D), jnp.float32),
            pltpu.VMEM((C, D), jnp.float32),
            pltpu.VMEM_SHARED((AGG_ROWS, D), jnp.float32),
            pltpu.SemaphoreType.DMA,
            pltpu.SemaphoreType.DMA,
            pltpu.SemaphoreType.DMA,
            pltpu.SemaphoreType.DMA,
        ],
    )
    def k(h_hbm, ep_hbm, src_hbm, dst_hbm, out_hbm,
          src_v, dst_v, hbuf0, hbuf1, epbuf0, epbuf1, agg,
          gsem0, gsem1, esem0, esem1):
        cid = lax.axis_index("c")
        sid = lax.axis_index("s")
        wid = cid * NSUB + sid
        row0 = wid * NCH
        nch = lax.min(NCH, NCHUNKS - row0)
        nblk = (nch + W_IDX - 1) // W_IDX

        def compute(hb, eb):
            @pl.loop(0, C)
            def _(e):
                for t in range(D // LANES):
                    sl = pl.ds(t * LANES, LANES)
                    hb[e, sl] = jnp.maximum(hb[e, sl] + eb[e, sl], 0.0)

        @pl.loop(0, C)
        def _(i):
            for t in range(D // LANES):
                hbuf0[i, pl.ds(t * LANES, LANES)] = jnp.zeros((LANES,), jnp.float32)

        for r in range(9):
            pltpu.sync_copy(hbuf0, agg.at[pl.ds(sid * ZROWS + r * C, C)])
        pltpu.sync_copy(hbuf0.at[pl.ds(0, ZROWS - 9 * C)],
                        agg.at[pl.ds(sid * ZROWS + 9 * C, ZROWS - 9 * C)])
        plsc.subcore_barrier()

        @pl.loop(0, nblk)
        def _(blk):
            base = blk * W_IDX
            nin = lax.min(W_IDX, nch - base)
            npair = nin // 2
            gstart = row0 + base
            cstart = (lax.min(gstart, NCHUNKS - W_READ) // 8) * 8
            shift = gstart - cstart
            pltpu.sync_copy(src_hbm.at[pl.ds(cstart, W_READ)], src_v)
            pltpu.sync_copy(dst_hbm.at[pl.ds(cstart, W_READ)], dst_v)
            pltpu.async_copy(h_hbm.at[src_v.at[shift]], hbuf0, gsem0)
            pltpu.async_copy(ep_hbm.at[gstart], epbuf0, esem0)

            @pl.loop(0, npair)
            def _(p):
                a = shift + 2 * p
                b = a + 1
                ga = gstart + 2 * p
                pltpu.make_async_copy(h_hbm.at[src_v.at[a]], hbuf0, gsem0).wait()
                pltpu.make_async_copy(ep_hbm.at[ga], epbuf0, esem0).wait()
                pltpu.async_copy(h_hbm.at[src_v.at[b]], hbuf1, gsem1)
                pltpu.async_copy(ep_hbm.at[ga + 1], epbuf1, esem1)
                compute(hbuf0, epbuf0)
                pltpu.sync_copy(hbuf0, agg.at[dst_v.at[a]], add=True)
                pltpu.make_async_copy(h_hbm.at[src_v.at[b]], hbuf1, gsem1).wait()
                pltpu.make_async_copy(ep_hbm.at[ga + 1], epbuf1, esem1).wait()

                @pl.when(2 * p + 2 < nin)
                def _():
                    pltpu.async_copy(h_hbm.at[src_v.at[a + 2]], hbuf0, gsem0)
                    pltpu.async_copy(ep_hbm.at[ga + 2], epbuf0, esem0)

                compute(hbuf1, epbuf1)
                pltpu.sync_copy(hbuf1, agg.at[dst_v.at[b]], add=True)

        plsc.subcore_barrier()
        pltpu.sync_copy(agg.at[pl.ds(sid * ZROWS, ZROWS)],
                        out_hbm.at[cid, pl.ds(sid * ZROWS, ZROWS)])

    return k(h, ep, src_p, dst_p)


def kernel(x, edge_index, env_edge_attr, act_edge_attr,
           We0, be0, W0, b0, We1, be1, W1, b1, We2, be2, W2, b2):
    src_p = edge_index[0].astype(jnp.int32).reshape(NCHUNKS, C)
    dst_p = edge_index[1].astype(jnp.int32).reshape(NCHUNKS, C)

    (ep0,) = _ep_tc(env_edge_attr, [(We0, be0)])
    ep1, ep2 = _ep_tc(act_edge_attr, [(We1, be1), (We2, be2)])

    h = x
    for i, (ep, W, b) in enumerate(((ep0, W0, b0), (ep1, W1, b1),
                                    (ep2, W2, b2))):
        last = i == 2
        aggp = _sc_layer(h, ep, src_p, dst_p)
        h = _update_tc(h, aggp, W, b, do_relu=not last)
    return h
